# baseline (speedup 1.0000x reference)
import functools

import jax
import jax.numpy as jnp
from jax import lax
from jax.experimental import pallas as pl
from jax.experimental.pallas import tpu as pltpu
from jax.experimental.pallas import tpu_sc as plsc

_NC, _NS, _L = 2, 16, 16
_NW = _NC * _NS
_C = 48
_D = 512


@functools.partial(jax.jit, static_argnums=(4, 5))
def _run(x_pheno, x_gene, src, dst, e_pad, b_per_w):
    n_chunks = b_per_w // _C
    n_pairs = n_chunks // 2
    mesh = plsc.VectorSubcoreMesh(core_axis_name="c", subcore_axis_name="s")

    @functools.partial(
        pl.kernel,
        mesh=mesh,
        compiler_params=pltpu.CompilerParams(needs_layout_passes=False),
        out_type=jax.ShapeDtypeStruct((e_pad,), jnp.float32),
        scratch_types=[
            pltpu.VMEM((2, _C), jnp.int32),
            pltpu.VMEM((2, _C), jnp.int32),
            pltpu.VMEM((2, _C, _D), jnp.float32),
            pltpu.VMEM((2, _C, _D), jnp.float32),
            pltpu.VMEM((_L * _L,), jnp.float32),
            pltpu.VMEM((2, _C), jnp.float32),
            pltpu.SemaphoreType.DMA,
            pltpu.SemaphoreType.DMA,
        ],
    )
    def k(xp_hbm, xg_hbm, src_hbm, dst_hbm, out_hbm,
          src_v, dst_v, bufa, bufb, accs, out_v, gsem0, gsem1):
        wid = lax.axis_index("s") * _NC + lax.axis_index("c")
        base = wid * b_per_w
        gsems = (gsem0, gsem1)

        def issue(c, slot):
            off = base + c * _C
            pltpu.sync_copy(src_hbm.at[pl.ds(off, _C)], src_v.at[slot])
            pltpu.sync_copy(dst_hbm.at[pl.ds(off, _C)], dst_v.at[slot])
            cp_a = pltpu.async_copy(
                xp_hbm.at[src_v.at[slot]], bufa.at[slot], gsems[slot])
            cp_b = pltpu.async_copy(
                xg_hbm.at[dst_v.at[slot]], bufb.at[slot], gsems[slot])
            return cp_a, cp_b

        def wait_gathers(cps):
            for cp in cps:
                cp.wait()

        lane_last = lax.iota(jnp.int32, _L) * _L + (_L - 1)

        def compute(slot, c):
            ra = bufa.at[slot]
            rb = bufb.at[slot]
            ov = out_v.at[slot]

            def group_body(g, carry2):
                def edge_body(t, carry3):
                    e = g * _L + t
                    acc = ra[e, pl.ds(0, _L)] * rb[e, pl.ds(0, _L)]
                    for j in range(1, _D // _L):
                        acc = acc + (ra[e, pl.ds(j * _L, _L)]
                                     * rb[e, pl.ds(j * _L, _L)])
                    accs[pl.ds(t * _L, _L)] = plsc.cumsum(acc)
                    return carry3

                lax.fori_loop(0, _L, edge_body, 0, unroll=False)
                r = plsc.load_gather(accs, [lane_last])
                ov[pl.ds(g * _L, _L)] = 1.0 / (1.0 + jnp.exp(-r))
                return carry2

            lax.fori_loop(0, _C // _L, group_body, 0, unroll=False)
            pltpu.sync_copy(ov, out_hbm.at[pl.ds(base + c * _C, _C)])

        def pair_body(i, carry):
            c0 = 2 * i
            cps0 = issue(c0, 0)
            cps1 = issue(c0 + 1, 1)
            wait_gathers(cps0)
            compute(0, c0)
            wait_gathers(cps1)
            compute(1, c0 + 1)
            return carry

        lax.fori_loop(0, n_pairs, pair_body, 0, unroll=False)

    return k(x_pheno, x_gene, src, dst)


def kernel(x_pheno, x_gene, edge_label_index):
    n_edges = edge_label_index.shape[1]
    chunk_pair = 2 * _C
    b_per_w = -(-n_edges // (_NW * chunk_pair)) * chunk_pair
    e_pad = b_per_w * _NW
    eli = edge_label_index.astype(jnp.int32)
    src = jnp.pad(eli[0], (0, e_pad - n_edges))
    dst = jnp.pad(eli[1], (0, e_pad - n_edges))
    out = _run(x_pheno, x_gene, src, dst, e_pad, b_per_w)
    return out[:n_edges]


# speedup vs baseline: 1.5960x; 1.5960x over previous
import functools

import jax
import jax.numpy as jnp
from jax import lax
from jax.experimental import pallas as pl
from jax.experimental.pallas import tpu as pltpu
from jax.experimental.pallas import tpu_sc as plsc

_NC, _NS, _L = 2, 16, 16
_NW = _NC * _NS
_C = 48
_D = 512


@functools.partial(jax.jit, static_argnums=(4, 5))
def _run(x_pheno, x_gene, src, dst, e_pad, b_per_w):
    n_chunks = b_per_w // _C
    n_pairs = n_chunks // 2
    mesh = plsc.VectorSubcoreMesh(core_axis_name="c", subcore_axis_name="s")

    @functools.partial(
        pl.kernel,
        mesh=mesh,
        compiler_params=pltpu.CompilerParams(needs_layout_passes=False),
        out_type=jax.ShapeDtypeStruct((e_pad,), jnp.float32),
        scratch_types=[
            pltpu.VMEM((b_per_w,), jnp.int32),
            pltpu.VMEM((b_per_w,), jnp.int32),
            pltpu.VMEM((2, _C, _D), jnp.float32),
            pltpu.VMEM((2, _C, _D), jnp.float32),
            pltpu.VMEM((_L * _L,), jnp.float32),
            pltpu.VMEM((2, _C), jnp.float32),
            pltpu.SemaphoreType.DMA,
            pltpu.SemaphoreType.DMA,
            pltpu.SemaphoreType.DMA,
            pltpu.SemaphoreType.DMA,
        ],
    )
    def k(xp_hbm, xg_hbm, src_hbm, dst_hbm, out_hbm,
          src_v, dst_v, bufa, bufb, accs, out_v, gsem0, gsem1, osem0, osem1):
        wid = lax.axis_index("s") * _NC + lax.axis_index("c")
        base = wid * b_per_w
        pltpu.sync_copy(src_hbm.at[pl.ds(base, b_per_w)], src_v)
        pltpu.sync_copy(dst_hbm.at[pl.ds(base, b_per_w)], dst_v)
        gsems = (gsem0, gsem1)
        osems = (osem0, osem1)

        def issue(c, slot):
            off = c * _C
            pltpu.async_copy(
                xp_hbm.at[src_v.at[pl.ds(off, _C)]], bufa.at[slot],
                gsems[slot])
            pltpu.async_copy(
                xg_hbm.at[dst_v.at[pl.ds(off, _C)]], bufb.at[slot],
                gsems[slot])

        def wait_gathers(slot):
            pltpu.make_async_copy(
                xp_hbm.at[src_v.at[pl.ds(0, _C)]], bufa.at[slot],
                gsems[slot]).wait()
            pltpu.make_async_copy(
                xg_hbm.at[dst_v.at[pl.ds(0, _C)]], bufb.at[slot],
                gsems[slot]).wait()

        def wait_writeback(slot):
            pltpu.make_async_copy(
                out_v.at[slot], out_hbm.at[pl.ds(base, _C)],
                osems[slot]).wait()

        def compute(slot, c, i):
            ra = bufa.at[slot]
            rb = bufb.at[slot]
            ov = out_v.at[slot]

            @pl.when(i > 0)
            def _():
                wait_writeback(slot)

            def group_body(g, carry2):
                def edge_body(t, carry3):
                    e = g * _L + t
                    acc = ra[e, pl.ds(0, _L)] * rb[e, pl.ds(0, _L)]
                    for j in range(1, _D // _L):
                        acc = acc + (ra[e, pl.ds(j * _L, _L)]
                                     * rb[e, pl.ds(j * _L, _L)])
                    accs[pl.ds(t * _L, _L)] = acc
                    return carry3

                lax.fori_loop(0, _L, edge_body, 0, unroll=False)
                row_base = lax.iota(jnp.int32, _L) * _L
                r = plsc.load_gather(accs, [row_base])
                for dcol in range(1, _L):
                    r = r + plsc.load_gather(accs, [row_base + dcol])
                ov[pl.ds(g * _L, _L)] = 1.0 / (1.0 + jnp.exp(-r))
                return carry2

            lax.fori_loop(0, _C // _L, group_body, 0, unroll=False)
            pltpu.async_copy(ov, out_hbm.at[pl.ds(base + c * _C, _C)],
                             osems[slot])

        issue(0, 0)

        def pair_body(i, carry):
            c0 = 2 * i
            issue(c0 + 1, 1)
            wait_gathers(0)
            compute(0, c0, i)
            issue(jnp.minimum(c0 + 2, n_chunks - 1), 0)
            wait_gathers(1)
            compute(1, c0 + 1, i)
            return carry

        lax.fori_loop(0, n_pairs, pair_body, 0, unroll=False)
        wait_gathers(0)
        wait_writeback(0)
        wait_writeback(1)

    return k(x_pheno, x_gene, src, dst)


def kernel(x_pheno, x_gene, edge_label_index):
    n_edges = edge_label_index.shape[1]
    chunk_pair = 2 * _C
    b_per_w = -(-n_edges // (_NW * chunk_pair)) * chunk_pair
    e_pad = b_per_w * _NW
    eli = edge_label_index.astype(jnp.int32)
    src = jnp.pad(eli[0], (0, e_pad - n_edges))
    dst = jnp.pad(eli[1], (0, e_pad - n_edges))
    out = _run(x_pheno, x_gene, src, dst, e_pad, b_per_w)
    return out[:n_edges]


# speedup vs baseline: 1.6373x; 1.0258x over previous
import functools

import jax
import jax.numpy as jnp
from jax import lax
from jax.experimental import pallas as pl
from jax.experimental.pallas import tpu as pltpu
from jax.experimental.pallas import tpu_sc as plsc

_NC, _NS, _L = 2, 16, 16
_NW = _NC * _NS
_C = 32
_D = 512
_NSLOT = 3


@functools.partial(jax.jit, static_argnums=(4, 5))
def _run(x_pheno, x_gene, src, dst, e_pad, b_per_w):
    n_chunks = b_per_w // _C
    n_rounds = n_chunks // _NSLOT
    mesh = plsc.VectorSubcoreMesh(core_axis_name="c", subcore_axis_name="s")

    @functools.partial(
        pl.kernel,
        mesh=mesh,
        compiler_params=pltpu.CompilerParams(needs_layout_passes=False),
        out_type=jax.ShapeDtypeStruct((e_pad,), jnp.float32),
        scratch_types=[
            pltpu.VMEM((b_per_w,), jnp.int32),
            pltpu.VMEM((b_per_w,), jnp.int32),
            pltpu.VMEM((_NSLOT, _C, _D), jnp.float32),
            pltpu.VMEM((_NSLOT, _C, _D), jnp.float32),
            pltpu.VMEM((_L * _L,), jnp.float32),
            pltpu.VMEM((_NSLOT, _C), jnp.float32),
            pltpu.SemaphoreType.DMA,
            pltpu.SemaphoreType.DMA,
            pltpu.SemaphoreType.DMA,
            pltpu.SemaphoreType.DMA,
            pltpu.SemaphoreType.DMA,
            pltpu.SemaphoreType.DMA,
        ],
    )
    def k(xp_hbm, xg_hbm, src_hbm, dst_hbm, out_hbm,
          src_v, dst_v, bufa, bufb, accs, out_v,
          gsem0, gsem1, gsem2, osem0, osem1, osem2):
        wid = lax.axis_index("s") * _NC + lax.axis_index("c")
        base = wid * b_per_w
        pltpu.sync_copy(src_hbm.at[pl.ds(base, b_per_w)], src_v)
        pltpu.sync_copy(dst_hbm.at[pl.ds(base, b_per_w)], dst_v)
        gsems = (gsem0, gsem1, gsem2)
        osems = (osem0, osem1, osem2)

        def issue(c, slot):
            off = c * _C
            pltpu.async_copy(
                xp_hbm.at[src_v.at[pl.ds(off, _C)]], bufa.at[slot],
                gsems[slot])
            pltpu.async_copy(
                xg_hbm.at[dst_v.at[pl.ds(off, _C)]], bufb.at[slot],
                gsems[slot])

        def wait_gathers(slot):
            pltpu.make_async_copy(
                xp_hbm.at[src_v.at[pl.ds(0, _C)]], bufa.at[slot],
                gsems[slot]).wait()
            pltpu.make_async_copy(
                xg_hbm.at[dst_v.at[pl.ds(0, _C)]], bufb.at[slot],
                gsems[slot]).wait()

        def wait_writeback(slot):
            pltpu.make_async_copy(
                out_v.at[slot], out_hbm.at[pl.ds(base, _C)],
                osems[slot]).wait()

        def compute(slot, c, i):
            ra = bufa.at[slot]
            rb = bufb.at[slot]
            ov = out_v.at[slot]

            @pl.when(i > 0)
            def _():
                wait_writeback(slot)

            def group_body(g, carry2):
                def edge_body(t, carry3):
                    e = g * _L + t
                    acc = ra[e, pl.ds(0, _L)] * rb[e, pl.ds(0, _L)]
                    for j in range(1, _D // _L):
                        acc = acc + (ra[e, pl.ds(j * _L, _L)]
                                     * rb[e, pl.ds(j * _L, _L)])
                    accs[pl.ds(t * _L, _L)] = acc
                    return carry3

                lax.fori_loop(0, _L, edge_body, 0, unroll=False)
                row_base = lax.iota(jnp.int32, _L) * _L
                r = plsc.load_gather(accs, [row_base])
                for dcol in range(1, _L):
                    r = r + plsc.load_gather(accs, [row_base + dcol])
                ov[pl.ds(g * _L, _L)] = 1.0 / (1.0 + jnp.exp(-r))
                return carry2

            lax.fori_loop(0, _C // _L, group_body, 0, unroll=False)
            pltpu.async_copy(ov, out_hbm.at[pl.ds(base + c * _C, _C)],
                             osems[slot])

        for b in range(_NSLOT):
            issue(b, b)

        def round_body(i, carry):
            c0 = _NSLOT * i
            for b in range(_NSLOT):
                wait_gathers(b)
                compute(b, c0 + b, i)
                issue(jnp.minimum(c0 + b + _NSLOT, n_chunks - 1), b)
            return carry

        lax.fori_loop(0, n_rounds, round_body, 0, unroll=False)
        for b in range(_NSLOT):
            wait_gathers(b)
            wait_writeback(b)

    return k(x_pheno, x_gene, src, dst)


def kernel(x_pheno, x_gene, edge_label_index):
    n_edges = edge_label_index.shape[1]
    chunk_round = _NSLOT * _C
    b_per_w = -(-n_edges // (_NW * chunk_round)) * chunk_round
    e_pad = b_per_w * _NW
    eli = edge_label_index.astype(jnp.int32)
    src = jnp.pad(eli[0], (0, e_pad - n_edges))
    dst = jnp.pad(eli[1], (0, e_pad - n_edges))
    out = _run(x_pheno, x_gene, src, dst, e_pad, b_per_w)
    return out[:n_edges]
